# baseline (speedup 1.0000x reference)
import jax
import jax.numpy as jnp
from jax.experimental import pallas as pl

B = 256
J = 17
S = 96 * 72
TOPK = 8
BB = 8


def _body(x_ref, y_ref, o_ref):
    i = pl.program_id(0)
    d = x_ref[...] - y_ref[...]
    s = jnp.sum(d * d, axis=2)
    l = s * (0.5 / S)

    jidx = jax.lax.broadcasted_iota(jnp.int32, (BB, J), 1)
    rank = jnp.zeros((BB, J), jnp.int32)
    for k in range(J):
        lk = l[:, k:k + 1]
        gt = (lk > l) | ((lk == l) & (k < jidx))
        rank = rank + gt.astype(jnp.int32)
    keep = rank < TOPK
    part = jnp.sum(jnp.where(keep, l, 0.0)) * (1.0 / (TOPK * B))

    @pl.when(i == 0)
    def _():
        o_ref[...] = jnp.zeros_like(o_ref)

    o_ref[...] = o_ref[...] + part[None, None]


def kernel(output, target):
    x = output.reshape(B, J, S)
    y = target.reshape(B, J, S)
    out = pl.pallas_call(
        _body,
        grid=(B // BB,),
        in_specs=[
            pl.BlockSpec((BB, J, S), lambda i: (i, 0, 0)),
            pl.BlockSpec((BB, J, S), lambda i: (i, 0, 0)),
        ],
        out_specs=pl.BlockSpec((1, 1), lambda i: (0, 0)),
        out_shape=jax.ShapeDtypeStruct((1, 1), jnp.float32),
    )(x, y)
    return out[0, 0]
